# baseline (speedup 1.0000x reference)
import functools
import jax
import jax.numpy as jnp
from jax import lax
from jax.experimental import pallas as pl
from jax.experimental.pallas import tpu as pltpu
from jax.experimental.pallas import tpu_sc as plsc


def _emb_lookup(indices, table, *, rows_per_w, chunk_r, nc):
    B, S = indices.shape
    D = table.shape[1]
    n_chunks = rows_per_w // chunk_r

    mesh = plsc.VectorSubcoreMesh(core_axis_name="c", subcore_axis_name="s")

    @functools.partial(
        pl.kernel,
        out_type=jax.ShapeDtypeStruct((B, S, D), jnp.float32),
        mesh=mesh,
        compiler_params=pltpu.CompilerParams(use_tc_tiling_on_sc=False),
        scratch_types=[
            pltpu.VMEM((2, chunk_r, S), jnp.int32),
            pltpu.VMEM((2, chunk_r, S, D), jnp.float32),
            pltpu.SemaphoreType.DMA((2,)),
            pltpu.SemaphoreType.DMA((2,)),
            pltpu.SemaphoreType.DMA((2,)),
        ],
    )
    def emb(idx_hbm, table_hbm, out_hbm, idx_v, rows_v, sem_i, sem_g, sem_o):
        wid = lax.axis_index("s") * nc + lax.axis_index("c")
        base = wid * rows_per_w

        def idx_copy(i, b):
            return pltpu.make_async_copy(
                idx_hbm.at[pl.ds(base + i * chunk_r, chunk_r)],
                idx_v.at[b],
                sem_i.at[b],
            )

        def gathers(b):
            return [
                pltpu.make_async_copy(
                    table_hbm.at[idx_v.at[b, j]],
                    rows_v.at[b, j],
                    sem_g.at[b],
                )
                for j in range(chunk_r)
            ]

        def out_copy(i, b):
            return pltpu.make_async_copy(
                rows_v.at[b],
                out_hbm.at[pl.ds(base + i * chunk_r, chunk_r)],
                sem_o.at[b],
            )

        idx_copy(0, 0).start()
        idx_copy(1, 1).start()

        def body(g, carry):
            for b in range(2):
                i = 2 * g + b
                idx_copy(i, b).wait()

                @pl.when(g > 0)
                def _():
                    out_copy(0, b).wait()

                gs = gathers(b)
                for d in gs:
                    d.start()
                for d in gs:
                    d.wait()

                @pl.when(i + 2 < n_chunks)
                def _():
                    idx_copy(i + 2, b).start()

                out_copy(i, b).start()
            return carry

        lax.fori_loop(0, n_chunks // 2, body, 0)
        out_copy(0, 0).wait()
        out_copy(0, 1).wait()

    return emb(indices, table)


def kernel(indices, table):
    B, S = indices.shape
    idx = indices.astype(jnp.int32)

    info = plsc.get_sparse_core_info()
    nc, ns = info.num_cores, info.num_subcores
    nw = nc * ns
    rows_per_w = B // nw

    return _emb_lookup(idx, table, rows_per_w=rows_per_w, chunk_r=8, nc=nc)


# speedup vs baseline: 1.0002x; 1.0002x over previous
import functools
import jax
import jax.numpy as jnp
from jax import lax
from jax.experimental import pallas as pl
from jax.experimental.pallas import tpu as pltpu
from jax.experimental.pallas import tpu_sc as plsc


def _emb_lookup(idx_flat, table, *, n_per_w, chunk, nc):
    N = idx_flat.shape[0]
    D = table.shape[1]
    n_chunks = n_per_w // chunk
    chunk_o = chunk * D // 128

    mesh = plsc.VectorSubcoreMesh(core_axis_name="c", subcore_axis_name="s")

    @functools.partial(
        pl.kernel,
        out_type=jax.ShapeDtypeStruct((N * D // 128, 128), jnp.float32),
        mesh=mesh,
        compiler_params=pltpu.CompilerParams(use_tc_tiling_on_sc=False),
        scratch_types=[
            pltpu.VMEM((2, chunk), jnp.int32),
            pltpu.VMEM((2, chunk, D), jnp.float32),
            pltpu.VMEM((2, chunk_o, 128), jnp.float32),
            pltpu.SemaphoreType.DMA((2,)),
            pltpu.SemaphoreType.DMA((2,)),
            pltpu.SemaphoreType.DMA((2,)),
        ],
    )
    def emb(idx_hbm, tab_hbm, out_hbm, idx_v, rows_v, out_v,
            sem_i, sem_g, sem_o):
        wid = lax.axis_index("s") * nc + lax.axis_index("c")
        base = wid * n_per_w
        obase = wid * (n_per_w * D // 128)

        def idx_copy(i, b):
            return pltpu.make_async_copy(
                idx_hbm.at[pl.ds(base + i * chunk, chunk)],
                idx_v.at[b],
                sem_i.at[b],
            )

        def gather(b):
            return pltpu.make_async_copy(
                tab_hbm.at[idx_v.at[b]],
                rows_v.at[b],
                sem_g.at[b],
            )

        def out_copy(i, b):
            return pltpu.make_async_copy(
                out_v.at[b],
                out_hbm.at[pl.ds(obase + i * chunk_o, chunk_o)],
                sem_o.at[b],
            )

        def repack(b):
            def step(g2, carry):
                r0 = g2 * 16
                l0 = g2 * 4
                for k in range(16):
                    lo = rows_v[b, r0 + k, pl.ds(0, 16)]
                    hi = rows_v[b, r0 + k, pl.ds(16, 16)]
                    out_v[b, l0 + k // 4, pl.ds((k % 4) * 32, 16)] = lo
                    out_v[b, l0 + k // 4, pl.ds((k % 4) * 32 + 16, 16)] = hi
                return carry

            lax.fori_loop(0, chunk // 16, step, 0)

        idx_copy(0, 0).start()
        idx_copy(1, 1).start()
        idx_copy(0, 0).wait()
        gather(0).start()

        def body(g, carry):
            for b in range(2):
                i = 2 * g + b
                nb = 1 - b
                gather(b).wait()

                @pl.when(i + 1 < n_chunks)
                def _():
                    idx_copy(i + 1, nb).wait()
                    gather(nb).start()

                @pl.when(g > 0)
                def _():
                    out_copy(0, b).wait()

                repack(b)

                @pl.when(i + 2 < n_chunks)
                def _():
                    idx_copy(i + 2, b).start()

                out_copy(i, b).start()
            return carry

        lax.fori_loop(0, n_chunks // 2, body, 0)
        out_copy(0, 0).wait()
        out_copy(0, 1).wait()

    return emb(idx_flat, table)


def kernel(indices, table):
    B, S = indices.shape
    D = table.shape[1]
    N = B * S

    info = plsc.get_sparse_core_info()
    nc, ns = info.num_cores, info.num_subcores
    nw = nc * ns
    n_per_w = N // nw

    idx_flat = indices.astype(jnp.int32).reshape(N)
    out128 = _emb_lookup(idx_flat, table, n_per_w=n_per_w, chunk=640, nc=nc)
    return out128.reshape(B, S, D)
